# baseline (speedup 1.0000x reference)
import functools

import jax
import jax.numpy as jnp
from jax.experimental import pallas as pl
from jax.experimental.pallas import tpu as pltpu

_B = 16384
_NH = 256
_NM = 1024
_NMPC = 8
_TEMP = 0.2
_EPS = 1e-4

_BLK = 256
_NBLK = _B // _BLK


def _loss_kernel(z_ref, y_ref, m_ref, acc_ref, msq_ref):
    i = pl.program_id(0)

    @pl.when(i == 0)
    def _():
        m = m_ref[...].astype(jnp.float32)
        msq_ref[...] = jnp.sum(m * m, axis=1, keepdims=True).T

    zb = z_ref[...].astype(jnp.float32)
    zsq = jnp.sum(zb * zb, axis=1, keepdims=True)
    xp = jax.lax.dot_general(
        z_ref[...], m_ref[...], (((1,), (1,)), ((), ())),
        preferred_element_type=jnp.float32,
    )
    d = zsq + msq_ref[...] - 2.0 * xp
    r = (d + 1.0) / (d + _EPS)
    r2 = r * r
    sim = r2 * r2 * r
    tot = jnp.sum(sim, axis=1)
    cls = jax.lax.broadcasted_iota(jnp.int32, (_BLK, _NM), 1) >> 3
    yb = y_ref[0, 0, :]
    pos = jnp.sum(jnp.where(cls == yb[:, None], sim, 0.0), axis=1)
    part = jnp.sum(jnp.log(tot) - jnp.log(pos)).reshape(1, 1)

    @pl.when(i == 0)
    def _():
        acc_ref[...] = part

    @pl.when(i != 0)
    def _():
        acc_ref[...] += part


def kernel(z, y, motif_vector):
    y3 = y.reshape(_NBLK, 1, _BLK)
    zh = z.astype(jnp.bfloat16)
    mh = motif_vector.astype(jnp.bfloat16)
    acc = pl.pallas_call(
        _loss_kernel,
        grid=(_NBLK,),
        in_specs=[
            pl.BlockSpec((_BLK, _NH), lambda i: (i, 0)),
            pl.BlockSpec((1, 1, _BLK), lambda i: (i, 0, 0)),
            pl.BlockSpec((_NM, _NH), lambda i: (0, 0)),
        ],
        out_specs=pl.BlockSpec((1, 1), lambda i: (0, 0)),
        out_shape=jax.ShapeDtypeStruct((1, 1), jnp.float32),
        scratch_shapes=[pltpu.VMEM((1, _NM), jnp.float32)],
    )(zh, y3, mh)
    return acc[0, 0] / _B


# speedup vs baseline: 1.2577x; 1.2577x over previous
import jax
import jax.numpy as jnp
from jax.experimental import pallas as pl
from jax.experimental.pallas import tpu as pltpu

_B = 16384
_NH = 256
_NM = 1024
_NC = 128
_NMPC = 8
_EPS = 1e-4

_BLK = 256
_NBLK = _B // _BLK


def _loss_kernel(z_ref, y_ref, m2_ref, acc_ref, msqe_ref):
    i = pl.program_id(0)

    @pl.when(i == 0)
    def _():
        m2 = m2_ref[...]
        msqe_ref[...] = 0.25 * jnp.sum(m2 * m2, axis=1, keepdims=True).T + _EPS

    zb = z_ref[...]
    zsq = jnp.sum(zb * zb, axis=1, keepdims=True)
    xp2 = jax.lax.dot_general(
        zb, m2_ref[...], (((1,), (1,)), ((), ())),
        preferred_element_type=jnp.float32,
    )
    den = zsq + msqe_ref[...] + xp2
    num = den + (1.0 - _EPS)
    r = num / den
    r2 = r * r
    sim = r2 * r2 * r
    persum = sim[:, 0:_NC]
    for k in range(1, _NMPC):
        persum = persum + sim[:, k * _NC:(k + 1) * _NC]
    tot = jnp.sum(persum, axis=1)
    cls = jax.lax.broadcasted_iota(jnp.int32, (_BLK, _NC), 1)
    yb = y_ref[0, 0, :]
    pos = jnp.sum(jnp.where(cls == yb[:, None], persum, 0.0), axis=1)
    part = jnp.sum(jnp.log(tot) - jnp.log(pos)).reshape(1, 1)

    @pl.when(i == 0)
    def _():
        acc_ref[...] = part

    @pl.when(i != 0)
    def _():
        acc_ref[...] += part


def kernel(z, y, motif_vector):
    y3 = y.reshape(_NBLK, 1, _BLK)
    m2 = -2.0 * motif_vector.reshape(_NC, _NMPC, _NH).transpose(1, 0, 2)
    m2 = m2.reshape(_NM, _NH)
    acc = pl.pallas_call(
        _loss_kernel,
        grid=(_NBLK,),
        in_specs=[
            pl.BlockSpec((_BLK, _NH), lambda i: (i, 0)),
            pl.BlockSpec((1, 1, _BLK), lambda i: (i, 0, 0)),
            pl.BlockSpec((_NM, _NH), lambda i: (0, 0)),
        ],
        out_specs=pl.BlockSpec((1, 1), lambda i: (0, 0)),
        out_shape=jax.ShapeDtypeStruct((1, 1), jnp.float32),
        scratch_shapes=[pltpu.VMEM((1, _NM), jnp.float32)],
    )(z, y3, m2)
    return acc[0, 0] / _B
